# baseline (speedup 1.0000x reference)
import functools

import jax
import jax.numpy as jnp
from jax import lax
from jax.experimental import pallas as pl
from jax.experimental.pallas import tpu as pltpu
from jax.experimental.pallas import tpu_sc as plsc

_N_TOKEN = 100000
_D_EMBED = 1024
_D_PROJ = 2048
_VEC_LEN = 128

_NC = 2
_NS = 16
_NW = _NC * _NS

_CHUNK = 64


def _sc_gather(table, idx_flat, n_tok):
  per_w = n_tok // _NW
  n_chunks = per_w // _CHUNK
  mesh = plsc.VectorSubcoreMesh(
      core_axis_name="c", subcore_axis_name="s",
      num_cores=_NC, num_subcores=_NS)

  @functools.partial(
      pl.kernel,
      out_type=jax.ShapeDtypeStruct((n_tok, _D_EMBED), jnp.float32),
      mesh=mesh,
      scratch_types=[
          pltpu.VMEM((_CHUNK,), jnp.int32),
          pltpu.VMEM((_CHUNK, _D_EMBED), jnp.float32),
          pltpu.SemaphoreType.DMA,
      ],
  )
  def gather_kernel(table_hbm, idx_hbm, out_hbm, idx_v, rows_v, sem):
    wid = lax.axis_index("s") * _NC + lax.axis_index("c")
    base = wid * per_w

    def body(i, carry):
      off = base + i * _CHUNK
      pltpu.sync_copy(idx_hbm.at[pl.ds(off, _CHUNK)], idx_v)
      pltpu.async_copy(table_hbm.at[idx_v], rows_v, sem).wait()
      pltpu.sync_copy(rows_v, out_hbm.at[pl.ds(off, _CHUNK)])
      return carry

    lax.fori_loop(0, n_chunks, body, 0)

  return gather_kernel(table, idx_flat)


def _proj_kernel(g_ref, s_ref, ws_ref, p_ref, o_ref):
  x = g_ref[...] + jnp.dot(
      s_ref[...], ws_ref[...], preferred_element_type=jnp.float32)
  acc = lax.dot_general(
      x, p_ref[...], (((1,), (1,)), ((), ())),
      preferred_element_type=jnp.float32)
  o_ref[...] = acc * (_D_PROJ ** 0.5)


def _tc_project(g, s_flat, status_weight, proj_W, n_tok, block_m):
  grid = (n_tok // block_m,)
  return pl.pallas_call(
      _proj_kernel,
      grid=grid,
      in_specs=[
          pl.BlockSpec((block_m, _D_EMBED), lambda i: (i, 0)),
          pl.BlockSpec((block_m, _VEC_LEN), lambda i: (i, 0)),
          pl.BlockSpec((_VEC_LEN, _D_EMBED), lambda i: (0, 0)),
          pl.BlockSpec((_D_PROJ, _D_EMBED), lambda i: (0, 0)),
      ],
      out_specs=pl.BlockSpec((block_m, _D_PROJ), lambda i: (i, 0)),
      out_shape=jax.ShapeDtypeStruct((n_tok, _D_PROJ), jnp.float32),
  )(g, s_flat, status_weight, proj_W)


def kernel(inp, status_vec, emb_weight, status_weight, proj_W):
  b, l = inp.shape
  n_tok = b * l
  idx_flat = inp.reshape(n_tok).astype(jnp.int32)
  g = _sc_gather(emb_weight, idx_flat, n_tok)
  s_flat = status_vec.reshape(n_tok, _VEC_LEN).astype(jnp.float32)
  out = _tc_project(g, s_flat, status_weight, proj_W, n_tok, block_m=512)
  return out.reshape(b, l, _D_PROJ)
